# baseline (speedup 1.0000x reference)
import functools

import jax
import jax.numpy as jnp
from jax import lax
from jax.experimental import pallas as pl
from jax.experimental.pallas import tpu as pltpu
from jax.experimental.pallas import tpu_sc as plsc

NC, NS = 2, 16
NW = NC * NS
D = 128

CS_UI = 128
CS_UU = 128
CD = 512
SUPD = 4 * CD

N_USER_K = 5000
N_TOTAL_K = 10000
E_UI_K = 320000
E_UU_K = 160000
N_ACC_UI = 10240
N_ACC_UU = 5120
N_CNT_UI = 10240
N_CNT_UU = 5120


def _fill(ref, rows, width, value):
    def row(i, _):
        for j in range(width // 16):
            ref[i, pl.ds(j * 16, 16)] = jnp.full((16,), value, jnp.float32)
        return 0
    lax.fori_loop(0, rows, row, 0)


def _make_prop(n_acc, n_edges, CS):
    SUP = 8 * CS
    n_sup = n_edges // SUP
    tail_off = n_sup * SUP
    tail_chunks = (n_edges - tail_off) // CS
    assert tail_off + tail_chunks * CS == n_edges
    iters = -(-n_sup // NW)
    rows_pt = n_acc // NS

    @functools.partial(
        pl.kernel,
        out_type=jax.ShapeDtypeStruct((NC * n_acc, D), jnp.float32),
        mesh=plsc.VectorSubcoreMesh(core_axis_name="c", subcore_axis_name="s"),
        compiler_params=pltpu.CompilerParams(use_tc_tiling_on_sc=False),
        scratch_types=[
            pltpu.VMEM_SHARED((n_acc, D), jnp.float32),
            pltpu.VMEM((SUP,), jnp.int32),
            pltpu.VMEM((SUP,), jnp.int32),
            pltpu.VMEM((SUP,), jnp.int32),
            pltpu.VMEM((SUP,), jnp.int32),
            pltpu.VMEM((CS, D), jnp.float32),
            pltpu.VMEM((CS, D), jnp.float32),
            pltpu.SemaphoreType.DMA,
            pltpu.SemaphoreType.DMA,
            pltpu.SemaphoreType.DMA,
            pltpu.SemaphoreType.DMA,
            pltpu.SemaphoreType.DMA,
            pltpu.SemaphoreType.DMA,
        ],
    )
    def prop(x_hbm, src_hbm, dst_hbm, out_hbm, acc_s, si0, si1, di0, di1,
             rb0, rb1, sg0, sg1, ss0, ss1, e0, e1):
        sid = lax.axis_index("s")
        cid = lax.axis_index("c")
        wid = sid * NC + cid
        rbs, sgs, sss = (rb0, rb1), (sg0, sg1), (ss0, ss1)
        sis, dis, eis = (si0, si1), (di0, di1), (e0, e1)
        _fill(rb0, CS, D, 0.0)
        base = pl.multiple_of(sid * rows_pt, 8)
        for k in range(rows_pt // CS):
            pltpu.sync_copy(rb0, acc_s.at[pl.ds(base + k * CS, CS)])
        if rows_pt % CS:
            rem = rows_pt % CS
            pltpu.sync_copy(rb0.at[pl.ds(0, rem)],
                            acc_s.at[pl.ds(base + (rows_pt // CS) * CS, rem)])
        plsc.subcore_barrier()

        nch = SUP // CS

        def fire_idx(sup, b):
            @pl.when(sup < n_sup)
            def _():
                off = pl.multiple_of(sup * SUP, SUP)
                pltpu.async_copy(src_hbm.at[pl.ds(off, SUP)], sis[b], eis[b])
                pltpu.async_copy(dst_hbm.at[pl.ds(off, SUP)], dis[b], eis[b])

        def process(sup, b):
            @pl.when(sup < n_sup)
            def _():
                pltpu.make_async_copy(
                    src_hbm.at[pl.ds(0, SUP)], sis[b], eis[b]).wait()
                pltpu.make_async_copy(
                    dst_hbm.at[pl.ds(0, SUP)], dis[b], eis[b]).wait()
                gd = [None] * nch
                sd = [None] * nch
                gd[0] = pltpu.async_copy(
                    x_hbm.at[sis[b].at[pl.ds(0, CS)]], rbs[0], sgs[0])
                for k in range(nch):
                    if k + 1 < nch:
                        if k >= 1:
                            sd[k - 1].wait()
                        gd[k + 1] = pltpu.async_copy(
                            x_hbm.at[sis[b].at[pl.ds((k + 1) * CS, CS)]],
                            rbs[(k + 1) % 2], sgs[(k + 1) % 2])
                    gd[k].wait()
                    sd[k] = pltpu.async_copy(
                        rbs[k % 2], acc_s.at[dis[b].at[pl.ds(k * CS, CS)]],
                        sss[k % 2], add=True)
                sd[nch - 2].wait()
                sd[nch - 1].wait()

        fire_idx(wid, 0)

        def body(j, _):
            s0 = wid + (2 * j) * NW
            s1 = wid + (2 * j + 1) * NW
            s2 = wid + (2 * j + 2) * NW
            fire_idx(s1, 1)
            process(s0, 0)
            fire_idx(s2, 0)
            process(s1, 1)
            return 0

        lax.fori_loop(0, -(-iters // 2), body, 0)

        @pl.when(wid == 0)
        def _():
            for t in range(tail_chunks):
                toff = tail_off + t * CS
                pltpu.sync_copy(src_hbm.at[pl.ds(toff, CS)],
                                si0.at[pl.ds(0, CS)])
                pltpu.sync_copy(dst_hbm.at[pl.ds(toff, CS)],
                                di0.at[pl.ds(0, CS)])
                pltpu.async_copy(
                    x_hbm.at[si0.at[pl.ds(0, CS)]], rb0, sg0).wait()
                pltpu.sync_copy(rb0, acc_s.at[di0.at[pl.ds(0, CS)]],
                                add=True)

        plsc.subcore_barrier()
        obase = pl.multiple_of(cid * n_acc + base, 8)
        pltpu.sync_copy(acc_s.at[pl.ds(base, rows_pt)],
                        out_hbm.at[pl.ds(obase, rows_pt)])

    return prop


@functools.partial(
    pl.kernel,
    out_type=tuple(
        jax.ShapeDtypeStruct((NC * n_acc, 16), jnp.float32)
        for n_acc in (N_CNT_UI, N_CNT_UI, N_CNT_UU, N_CNT_UU)
    ),
    mesh=plsc.VectorSubcoreMesh(core_axis_name="c", subcore_axis_name="s"),
    compiler_params=pltpu.CompilerParams(use_tc_tiling_on_sc=False),
    scratch_types=[
        pltpu.VMEM_SHARED((N_CNT_UI, 16), jnp.float32),
        pltpu.VMEM_SHARED((N_CNT_UI, 16), jnp.float32),
        pltpu.VMEM_SHARED((N_CNT_UU, 16), jnp.float32),
        pltpu.VMEM_SHARED((N_CNT_UU, 16), jnp.float32),
        pltpu.VMEM((SUPD,), jnp.int32),
        pltpu.VMEM((SUPD,), jnp.int32),
        pltpu.VMEM((CD, 16), jnp.float32),
        pltpu.VMEM((160, 16), jnp.float32),
        pltpu.SemaphoreType.DMA,
    ],
)
def _sc_degrees(s_ui, d_ui, s_uu, d_uu, o0, o1, o2, o3,
                cnt0, cnt1, cnt2, cnt3, is_v, id_v, ones_v, zb, sem):
    sid = lax.axis_index("s")
    cid = lax.axis_index("c")
    wid = sid * NC + cid
    cnts = (cnt0, cnt1, cnt2, cnt3)
    outs = (o0, o1, o2, o3)
    sizes = (N_CNT_UI, N_CNT_UI, N_CNT_UU, N_CNT_UU)
    _fill(zb, 160, 16, 0.0)
    _fill(ones_v, CD, 16, 1.0)
    for cnt, n_acc in zip(cnts, sizes):
        rpt = n_acc // NS
        base = pl.multiple_of(sid * rpt, 8)
        for k in range(rpt // 160):
            pltpu.sync_copy(zb, cnt.at[pl.ds(base + k * 160, 160)])
    plsc.subcore_barrier()

    for src_hbm, dst_hbm, cnt_s, cnt_d, n_edges in (
        (s_ui, d_ui, cnt0, cnt1, E_UI_K),
        (s_uu, d_uu, cnt2, cnt3, E_UU_K),
    ):
        n_sup = n_edges // SUPD
        tail_off = n_sup * SUPD
        tail = n_edges - tail_off
        iters = -(-n_sup // NW)

        def body(it, _, src_hbm=src_hbm, dst_hbm=dst_hbm,
                 cnt_s=cnt_s, cnt_d=cnt_d, n_sup=n_sup):
            sup = wid + it * NW

            @pl.when(sup < n_sup)
            def _():
                off = pl.multiple_of(sup * SUPD, SUPD)
                pltpu.sync_copy(src_hbm.at[pl.ds(off, SUPD)], is_v)
                pltpu.sync_copy(dst_hbm.at[pl.ds(off, SUPD)], id_v)
                ds = []
                for k in range(SUPD // CD):
                    ds.append(pltpu.async_copy(
                        ones_v, cnt_s.at[is_v.at[pl.ds(k * CD, CD)]],
                        sem, add=True))
                    ds.append(pltpu.async_copy(
                        ones_v, cnt_d.at[id_v.at[pl.ds(k * CD, CD)]],
                        sem, add=True))
                for d in ds:
                    d.wait()
            return 0

        lax.fori_loop(0, iters, body, 0)

        @pl.when(wid == 0)
        def _(src_hbm=src_hbm, dst_hbm=dst_hbm, cnt_s=cnt_s, cnt_d=cnt_d,
              tail_off=tail_off, tail=tail):
            pltpu.sync_copy(src_hbm.at[pl.ds(tail_off, tail)],
                            is_v.at[pl.ds(0, tail)])
            pltpu.sync_copy(dst_hbm.at[pl.ds(tail_off, tail)],
                            id_v.at[pl.ds(0, tail)])
            pltpu.sync_copy(ones_v.at[pl.ds(0, tail)],
                            cnt_s.at[is_v.at[pl.ds(0, tail)]], add=True)
            pltpu.sync_copy(ones_v.at[pl.ds(0, tail)],
                            cnt_d.at[id_v.at[pl.ds(0, tail)]], add=True)

    plsc.subcore_barrier()
    for cnt, out, n_acc in zip(cnts, outs, sizes):
        rpt = n_acc // NS
        base = pl.multiple_of(sid * rpt, 8)
        obase = pl.multiple_of(cid * n_acc + base, 8)
        pltpu.sync_copy(cnt.at[pl.ds(base, rpt)],
                        out.at[pl.ds(obase, rpt)])


def _tc_prep(n_nodes, r=1000):
    def body(x_ref, cs_ref, o_ref):
        cs = cs_ref[...]
        c = cs[0, :, 0:1] + cs[1, :, 0:1]
        o_ref[...] = x_ref[...] * lax.rsqrt(jnp.maximum(c, 1.0))

    return pl.pallas_call(
        body,
        grid=(n_nodes // r,),
        in_specs=[pl.BlockSpec((r, D), lambda i: (i, 0)),
                  pl.BlockSpec((2, r, 16), lambda i: (0, i, 0))],
        out_specs=pl.BlockSpec((r, D), lambda i: (i, 0)),
        out_shape=jax.ShapeDtypeStruct((n_nodes, D), jnp.float32),
    )


def _tc_layer(n_nodes, r=1000):
    def body(p_ref, cs_ref, cd_ref, a_ref, ao_ref, xs_ref):
        p = p_ref[...]
        y = p[0] + p[1]
        cd = cd_ref[...]
        c_in = cd[0, :, 0:1] + cd[1, :, 0:1]
        y = y * lax.rsqrt(jnp.maximum(c_in, 1.0))
        y = jnp.where(y >= 0, y, 0.5 * y)
        ss = jnp.sum(y * y, axis=1, keepdims=True)
        ao_ref[...] = a_ref[...] + y / jnp.maximum(jnp.sqrt(ss), 1e-12)
        cs = cs_ref[...]
        c_out = cs[0, :, 0:1] + cs[1, :, 0:1]
        xs_ref[...] = y * lax.rsqrt(jnp.maximum(c_out, 1.0))

    return pl.pallas_call(
        body,
        grid=(n_nodes // r,),
        in_specs=[pl.BlockSpec((2, r, D), lambda i: (0, i, 0)),
                  pl.BlockSpec((2, r, 16), lambda i: (0, i, 0)),
                  pl.BlockSpec((2, r, 16), lambda i: (0, i, 0)),
                  pl.BlockSpec((r, D), lambda i: (i, 0))],
        out_specs=[pl.BlockSpec((r, D), lambda i: (i, 0))] * 2,
        out_shape=[jax.ShapeDtypeStruct((n_nodes, D), jnp.float32)] * 2,
    )


_prop_ui = _make_prop(N_ACC_UI, E_UI_K, CS_UI)
_prop_uu = _make_prop(N_ACC_UU, E_UU_K, CS_UU)
_prep_all = _tc_prep(N_TOTAL_K + N_USER_K)
_layer_all = _tc_layer(N_TOTAL_K + N_USER_K)


def kernel(user_emb, item_emb, edge_index_ui, edge_index_uu):
    su, du = edge_index_ui[0], edge_index_ui[1]
    sv, dv = edge_index_uu[0], edge_index_uu[1]

    c_su, c_du, c_sv, c_dv = _sc_degrees(su, du, sv, dv)
    c_su = c_su.reshape(NC, N_CNT_UI, 16)[:, :N_TOTAL_K]
    c_du = c_du.reshape(NC, N_CNT_UI, 16)[:, :N_TOTAL_K]
    c_sv = c_sv.reshape(NC, N_CNT_UU, 16)[:, :N_USER_K]
    c_dv = c_dv.reshape(NC, N_CNT_UU, 16)[:, :N_USER_K]
    c_s = jnp.concatenate([c_su, c_sv], axis=1)
    c_d = jnp.concatenate([c_du, c_dv], axis=1)

    NT = N_TOTAL_K + N_USER_K
    x0 = jnp.concatenate([user_emb, item_emb, user_emb], axis=0)
    xs = _prep_all(x0, c_s)
    for _ in range(2):
        p_ui = _prop_ui(xs[:N_TOTAL_K], su, du)
        p_uu = _prop_uu(xs[N_TOTAL_K:], sv, dv)
        p = jnp.concatenate(
            [p_ui.reshape(NC, N_ACC_UI, D)[:, :N_TOTAL_K],
             p_uu.reshape(NC, N_ACC_UU, D)[:, :N_USER_K]], axis=1)
        x0, xs = _layer_all(p, c_s, c_d, x0)

    return (x0[:N_TOTAL_K], x0[N_TOTAL_K:])


# speedup vs baseline: 1.1643x; 1.1643x over previous
import functools

import jax
import jax.numpy as jnp
from jax import lax
from jax.experimental import pallas as pl
from jax.experimental.pallas import tpu as pltpu
from jax.experimental.pallas import tpu_sc as plsc

NC, NS = 2, 16
NW = NC * NS
D = 128

CS_UI = 128
CS_UU = 128
CD = 512
SUPD = 4 * CD

N_USER_K = 5000
N_TOTAL_K = 10000
E_UI_K = 320000
E_UU_K = 160000
N_ACC_UI = 10240
N_ACC_UU = 5120
N_CNT_UI = 10240
N_CNT_UU = 5120


def _fill(ref, rows, width, value):
    def row(i, _):
        for j in range(width // 16):
            ref[i, pl.ds(j * 16, 16)] = jnp.full((16,), value, jnp.float32)
        return 0
    lax.fori_loop(0, rows, row, 0)


def _make_prop(n_acc, n_edges, CS):
    SUP = 8 * CS
    n_sup = n_edges // SUP
    tail_off = n_sup * SUP
    tail_chunks = (n_edges - tail_off) // CS
    assert tail_off + tail_chunks * CS == n_edges
    iters = -(-n_sup // NW)
    rows_pt = n_acc // NS

    @functools.partial(
        pl.kernel,
        out_type=jax.ShapeDtypeStruct((NC * n_acc, D), jnp.float32),
        mesh=plsc.VectorSubcoreMesh(core_axis_name="c", subcore_axis_name="s"),
        compiler_params=pltpu.CompilerParams(use_tc_tiling_on_sc=False),
        scratch_types=[
            pltpu.VMEM_SHARED((n_acc, D), jnp.float32),
            pltpu.VMEM((SUP,), jnp.int32),
            pltpu.VMEM((SUP,), jnp.int32),
            pltpu.VMEM((SUP,), jnp.int32),
            pltpu.VMEM((SUP,), jnp.int32),
            pltpu.VMEM((CS, D), jnp.float32),
            pltpu.VMEM((CS, D), jnp.float32),
            pltpu.SemaphoreType.DMA,
            pltpu.SemaphoreType.DMA,
            pltpu.SemaphoreType.DMA,
            pltpu.SemaphoreType.DMA,
            pltpu.SemaphoreType.DMA,
            pltpu.SemaphoreType.DMA,
        ],
    )
    def prop(x_hbm, src_hbm, dst_hbm, out_hbm, acc_s, si0, si1, di0, di1,
             rb0, rb1, sg0, sg1, ss0, ss1, e0, e1):
        sid = lax.axis_index("s")
        cid = lax.axis_index("c")
        wid = sid * NC + cid
        rbs, sgs, sss = (rb0, rb1), (sg0, sg1), (ss0, ss1)
        sis, dis, eis = (si0, si1), (di0, di1), (e0, e1)
        _fill(rb0, CS, D, 0.0)
        base = pl.multiple_of(sid * rows_pt, 8)
        for k in range(rows_pt // CS):
            pltpu.sync_copy(rb0, acc_s.at[pl.ds(base + k * CS, CS)])
        if rows_pt % CS:
            rem = rows_pt % CS
            pltpu.sync_copy(rb0.at[pl.ds(0, rem)],
                            acc_s.at[pl.ds(base + (rows_pt // CS) * CS, rem)])
        plsc.subcore_barrier()

        nch = SUP // CS

        def fire_idx(sup, b):
            @pl.when(sup < n_sup)
            def _():
                off = pl.multiple_of(sup * SUP, SUP)
                pltpu.async_copy(src_hbm.at[pl.ds(off, SUP)], sis[b], eis[b])
                pltpu.async_copy(dst_hbm.at[pl.ds(off, SUP)], dis[b], eis[b])

        def process(sup, b):
            @pl.when(sup < n_sup)
            def _():
                pltpu.make_async_copy(
                    src_hbm.at[pl.ds(0, SUP)], sis[b], eis[b]).wait()
                pltpu.make_async_copy(
                    dst_hbm.at[pl.ds(0, SUP)], dis[b], eis[b]).wait()
                gd = [None] * nch
                sd = [None] * nch
                gd[0] = pltpu.async_copy(
                    x_hbm.at[sis[b].at[pl.ds(0, CS)]], rbs[0], sgs[0])
                for k in range(nch):
                    if k + 1 < nch:
                        if k >= 1:
                            sd[k - 1].wait()
                        gd[k + 1] = pltpu.async_copy(
                            x_hbm.at[sis[b].at[pl.ds((k + 1) * CS, CS)]],
                            rbs[(k + 1) % 2], sgs[(k + 1) % 2])
                    gd[k].wait()
                    sd[k] = pltpu.async_copy(
                        rbs[k % 2], acc_s.at[dis[b].at[pl.ds(k * CS, CS)]],
                        sss[k % 2], add=True)
                sd[nch - 2].wait()
                sd[nch - 1].wait()

        fire_idx(wid, 0)

        def body(j, _):
            s0 = wid + (2 * j) * NW
            s1 = wid + (2 * j + 1) * NW
            s2 = wid + (2 * j + 2) * NW
            fire_idx(s1, 1)
            process(s0, 0)
            fire_idx(s2, 0)
            process(s1, 1)
            return 0

        lax.fori_loop(0, -(-iters // 2), body, 0)

        @pl.when(wid == 0)
        def _():
            for t in range(tail_chunks):
                toff = tail_off + t * CS
                pltpu.sync_copy(src_hbm.at[pl.ds(toff, CS)],
                                si0.at[pl.ds(0, CS)])
                pltpu.sync_copy(dst_hbm.at[pl.ds(toff, CS)],
                                di0.at[pl.ds(0, CS)])
                pltpu.async_copy(
                    x_hbm.at[si0.at[pl.ds(0, CS)]], rb0, sg0).wait()
                pltpu.sync_copy(rb0, acc_s.at[di0.at[pl.ds(0, CS)]],
                                add=True)

        plsc.subcore_barrier()
        obase = pl.multiple_of(cid * n_acc + base, 8)
        pltpu.sync_copy(acc_s.at[pl.ds(base, rows_pt)],
                        out_hbm.at[pl.ds(obase, rows_pt)])

    return prop


@functools.partial(
    pl.kernel,
    out_type=tuple(
        jax.ShapeDtypeStruct((NC * n_acc, 16), jnp.float32)
        for n_acc in (N_CNT_UI, N_CNT_UI, N_CNT_UU, N_CNT_UU)
    ),
    mesh=plsc.VectorSubcoreMesh(core_axis_name="c", subcore_axis_name="s"),
    compiler_params=pltpu.CompilerParams(use_tc_tiling_on_sc=False),
    scratch_types=[
        pltpu.VMEM_SHARED((N_CNT_UI, 16), jnp.float32),
        pltpu.VMEM_SHARED((N_CNT_UI, 16), jnp.float32),
        pltpu.VMEM_SHARED((N_CNT_UU, 16), jnp.float32),
        pltpu.VMEM_SHARED((N_CNT_UU, 16), jnp.float32),
        pltpu.VMEM((SUPD,), jnp.int32),
        pltpu.VMEM((SUPD,), jnp.int32),
        pltpu.VMEM((CD, 16), jnp.float32),
        pltpu.VMEM((160, 16), jnp.float32),
        pltpu.SemaphoreType.DMA,
    ],
)
def _sc_degrees(s_ui, d_ui, s_uu, d_uu, o0, o1, o2, o3,
                cnt0, cnt1, cnt2, cnt3, is_v, id_v, ones_v, zb, sem):
    sid = lax.axis_index("s")
    cid = lax.axis_index("c")
    wid = sid * NC + cid
    cnts = (cnt0, cnt1, cnt2, cnt3)
    outs = (o0, o1, o2, o3)
    sizes = (N_CNT_UI, N_CNT_UI, N_CNT_UU, N_CNT_UU)
    _fill(zb, 160, 16, 0.0)
    _fill(ones_v, CD, 16, 1.0)
    for cnt, n_acc in zip(cnts, sizes):
        rpt = n_acc // NS
        base = pl.multiple_of(sid * rpt, 8)
        for k in range(rpt // 160):
            pltpu.sync_copy(zb, cnt.at[pl.ds(base + k * 160, 160)])
    plsc.subcore_barrier()

    for src_hbm, dst_hbm, cnt_s, cnt_d, n_edges in (
        (s_ui, d_ui, cnt0, cnt1, E_UI_K),
        (s_uu, d_uu, cnt2, cnt3, E_UU_K),
    ):
        n_sup = n_edges // SUPD
        tail_off = n_sup * SUPD
        tail = n_edges - tail_off
        iters = -(-n_sup // NW)

        def body(it, _, src_hbm=src_hbm, dst_hbm=dst_hbm,
                 cnt_s=cnt_s, cnt_d=cnt_d, n_sup=n_sup):
            sup = wid + it * NW

            @pl.when(sup < n_sup)
            def _():
                off = pl.multiple_of(sup * SUPD, SUPD)
                pltpu.sync_copy(src_hbm.at[pl.ds(off, SUPD)], is_v)
                pltpu.sync_copy(dst_hbm.at[pl.ds(off, SUPD)], id_v)
                ds = []
                for k in range(SUPD // CD):
                    ds.append(pltpu.async_copy(
                        ones_v, cnt_s.at[is_v.at[pl.ds(k * CD, CD)]],
                        sem, add=True))
                    ds.append(pltpu.async_copy(
                        ones_v, cnt_d.at[id_v.at[pl.ds(k * CD, CD)]],
                        sem, add=True))
                for d in ds:
                    d.wait()
            return 0

        lax.fori_loop(0, iters, body, 0)

        @pl.when(wid == 0)
        def _(src_hbm=src_hbm, dst_hbm=dst_hbm, cnt_s=cnt_s, cnt_d=cnt_d,
              tail_off=tail_off, tail=tail):
            pltpu.sync_copy(src_hbm.at[pl.ds(tail_off, tail)],
                            is_v.at[pl.ds(0, tail)])
            pltpu.sync_copy(dst_hbm.at[pl.ds(tail_off, tail)],
                            id_v.at[pl.ds(0, tail)])
            pltpu.sync_copy(ones_v.at[pl.ds(0, tail)],
                            cnt_s.at[is_v.at[pl.ds(0, tail)]], add=True)
            pltpu.sync_copy(ones_v.at[pl.ds(0, tail)],
                            cnt_d.at[id_v.at[pl.ds(0, tail)]], add=True)

    plsc.subcore_barrier()
    for cnt, out, n_acc in zip(cnts, outs, sizes):
        rpt = n_acc // NS
        base = pl.multiple_of(sid * rpt, 8)
        obase = pl.multiple_of(cid * n_acc + base, 8)
        pltpu.sync_copy(cnt.at[pl.ds(base, rpt)],
                        out.at[pl.ds(obase, rpt)])


def _tc_prep(n_nodes, r=1000):
    def body(x_ref, cs_ref, o_ref):
        cs = cs_ref[...]
        c = cs[0, :, 0:1] + cs[1, :, 0:1]
        o_ref[...] = x_ref[...] * lax.rsqrt(jnp.maximum(c, 1.0))

    return pl.pallas_call(
        body,
        grid=(n_nodes // r,),
        in_specs=[pl.BlockSpec((r, D), lambda i: (i, 0)),
                  pl.BlockSpec((2, r, 16), lambda i: (0, i, 0))],
        out_specs=pl.BlockSpec((r, D), lambda i: (i, 0)),
        out_shape=jax.ShapeDtypeStruct((n_nodes, D), jnp.float32),
    )


def _tc_layer(n_nodes, r=1000):
    def body(p_ref, cs_ref, cd_ref, a_ref, ao_ref, xs_ref):
        p = p_ref[...]
        y = p[0] + p[1]
        cd = cd_ref[...]
        c_in = cd[0, :, 0:1] + cd[1, :, 0:1]
        y = y * lax.rsqrt(jnp.maximum(c_in, 1.0))
        y = jnp.where(y >= 0, y, 0.5 * y)
        ss = jnp.sum(y * y, axis=1, keepdims=True)
        ao_ref[...] = a_ref[...] + y / jnp.maximum(jnp.sqrt(ss), 1e-12)
        cs = cs_ref[...]
        c_out = cs[0, :, 0:1] + cs[1, :, 0:1]
        xs_ref[...] = y * lax.rsqrt(jnp.maximum(c_out, 1.0))

    return pl.pallas_call(
        body,
        grid=(n_nodes // r,),
        in_specs=[pl.BlockSpec((2, r, D), lambda i: (0, i, 0)),
                  pl.BlockSpec((2, r, 16), lambda i: (0, i, 0)),
                  pl.BlockSpec((2, r, 16), lambda i: (0, i, 0)),
                  pl.BlockSpec((r, D), lambda i: (i, 0))],
        out_specs=[pl.BlockSpec((r, D), lambda i: (i, 0))] * 2,
        out_shape=[jax.ShapeDtypeStruct((n_nodes, D), jnp.float32)] * 2,
    )


_prop_ui = _make_prop(N_ACC_UI, E_UI_K, CS_UI)
_prop_uu = _make_prop(N_ACC_UU, E_UU_K, CS_UU)
_prep_ui = _tc_prep(N_TOTAL_K)
_prep_uu = _tc_prep(N_USER_K)
_layer_ui = _tc_layer(N_TOTAL_K)
_layer_uu = _tc_layer(N_USER_K)


def kernel(user_emb, item_emb, edge_index_ui, edge_index_uu):
    su, du = edge_index_ui[0], edge_index_ui[1]
    sv, dv = edge_index_uu[0], edge_index_uu[1]

    c_su, c_du, c_sv, c_dv = _sc_degrees(su, du, sv, dv)
    c_su = c_su.reshape(NC, N_CNT_UI, 16)[:, :N_TOTAL_K]
    c_du = c_du.reshape(NC, N_CNT_UI, 16)[:, :N_TOTAL_K]
    c_sv = c_sv.reshape(NC, N_CNT_UU, 16)[:, :N_USER_K]
    c_dv = c_dv.reshape(NC, N_CNT_UU, 16)[:, :N_USER_K]

    x0 = jnp.concatenate([user_emb, item_emb], axis=0)
    xs = _prep_ui(x0, c_su)
    p = _prop_ui(xs, su, du).reshape(NC, N_ACC_UI, D)[:, :N_TOTAL_K]
    acc, xs = _layer_ui(p, c_su, c_du, x0)
    p = _prop_ui(xs, su, du).reshape(NC, N_ACC_UI, D)[:, :N_TOTAL_K]
    ui_out, _ = _layer_ui(p, c_su, c_du, acc)

    xs = _prep_uu(user_emb, c_sv)
    p = _prop_uu(xs, sv, dv).reshape(NC, N_ACC_UU, D)[:, :N_USER_K]
    acc, xs = _layer_uu(p, c_sv, c_dv, user_emb)
    p = _prop_uu(xs, sv, dv).reshape(NC, N_ACC_UU, D)[:, :N_USER_K]
    uu_out, _ = _layer_uu(p, c_sv, c_dv, acc)

    return (ui_out, uu_out)


# speedup vs baseline: 1.2305x; 1.0569x over previous
import functools

import jax
import jax.numpy as jnp
from jax import lax
from jax.experimental import pallas as pl
from jax.experimental.pallas import tpu as pltpu
from jax.experimental.pallas import tpu_sc as plsc

NC, NS = 2, 16
NW = NC * NS
D = 128

CS_UI = 128
CS_UU = 128
CD = 512
SUPD = 4 * CD

N_USER_K = 5000
N_TOTAL_K = 10000
E_UI_K = 320000
E_UU_K = 160000
N_ACC_UI = 10240
N_ACC_UU = 5120
N_CNT_UI = 10240
N_CNT_UU = 5120


def _fill(ref, rows, width, value):
    def row(i, _):
        for j in range(width // 16):
            ref[i, pl.ds(j * 16, 16)] = jnp.full((16,), value, jnp.float32)
        return 0
    lax.fori_loop(0, rows, row, 0)


def _make_prop(n_acc, n_edges, CS):
    SUP = 8 * CS
    n_sup = n_edges // SUP
    tail_off = n_sup * SUP
    tail_chunks = (n_edges - tail_off) // CS
    assert tail_off + tail_chunks * CS == n_edges
    iters = -(-n_sup // NW)
    rows_pt = n_acc // NS

    @functools.partial(
        pl.kernel,
        out_type=jax.ShapeDtypeStruct((NC * n_acc, D), jnp.float32),
        mesh=plsc.VectorSubcoreMesh(core_axis_name="c", subcore_axis_name="s"),
        compiler_params=pltpu.CompilerParams(use_tc_tiling_on_sc=False),
        scratch_types=[
            pltpu.VMEM_SHARED((n_acc, D), jnp.float32),
            pltpu.VMEM((SUP,), jnp.int32),
            pltpu.VMEM((SUP,), jnp.int32),
            pltpu.VMEM((SUP,), jnp.int32),
            pltpu.VMEM((SUP,), jnp.int32),
            pltpu.VMEM((CS, D), jnp.float32),
            pltpu.VMEM((CS, D), jnp.float32),
            pltpu.SemaphoreType.DMA,
            pltpu.SemaphoreType.DMA,
            pltpu.SemaphoreType.DMA,
            pltpu.SemaphoreType.DMA,
            pltpu.SemaphoreType.DMA,
            pltpu.SemaphoreType.DMA,
        ],
    )
    def prop(x_hbm, src_hbm, dst_hbm, out_hbm, acc_s, si0, si1, di0, di1,
             rb0, rb1, sg0, sg1, ss0, ss1, e0, e1):
        sid = lax.axis_index("s")
        cid = lax.axis_index("c")
        wid = sid * NC + cid
        rbs, sgs, sss = (rb0, rb1), (sg0, sg1), (ss0, ss1)
        sis, dis, eis = (si0, si1), (di0, di1), (e0, e1)
        _fill(rb0, CS, D, 0.0)
        base = pl.multiple_of(sid * rows_pt, 8)
        for k in range(rows_pt // CS):
            pltpu.sync_copy(rb0, acc_s.at[pl.ds(base + k * CS, CS)])
        if rows_pt % CS:
            rem = rows_pt % CS
            pltpu.sync_copy(rb0.at[pl.ds(0, rem)],
                            acc_s.at[pl.ds(base + (rows_pt // CS) * CS, rem)])
        plsc.subcore_barrier()

        nch = SUP // CS

        def fire_idx(sup, b):
            @pl.when(sup < n_sup)
            def _():
                off = pl.multiple_of(sup * SUP, SUP)
                pltpu.async_copy(src_hbm.at[pl.ds(off, SUP)], sis[b], eis[b])
                pltpu.async_copy(dst_hbm.at[pl.ds(off, SUP)], dis[b], eis[b])

        def process(sup, b):
            @pl.when(sup < n_sup)
            def _():
                pltpu.make_async_copy(
                    src_hbm.at[pl.ds(0, SUP)], sis[b], eis[b]).wait()
                pltpu.make_async_copy(
                    dst_hbm.at[pl.ds(0, SUP)], dis[b], eis[b]).wait()
                gd = [None] * nch
                sd = [None] * nch
                gd[0] = pltpu.async_copy(
                    x_hbm.at[sis[b].at[pl.ds(0, CS)]], rbs[0], sgs[0])
                for k in range(nch):
                    if k + 1 < nch:
                        if k >= 1:
                            sd[k - 1].wait()
                        gd[k + 1] = pltpu.async_copy(
                            x_hbm.at[sis[b].at[pl.ds((k + 1) * CS, CS)]],
                            rbs[(k + 1) % 2], sgs[(k + 1) % 2])
                    gd[k].wait()
                    sd[k] = pltpu.async_copy(
                        rbs[k % 2], acc_s.at[dis[b].at[pl.ds(k * CS, CS)]],
                        sss[k % 2], add=True)
                sd[nch - 2].wait()
                sd[nch - 1].wait()

        fire_idx(wid, 0)

        def body(j, _):
            s0 = wid + (2 * j) * NW
            s1 = wid + (2 * j + 1) * NW
            s2 = wid + (2 * j + 2) * NW
            fire_idx(s1, 1)
            process(s0, 0)
            fire_idx(s2, 0)
            process(s1, 1)
            return 0

        lax.fori_loop(0, -(-iters // 2), body, 0)

        @pl.when(wid == 0)
        def _():
            for t in range(tail_chunks):
                toff = tail_off + t * CS
                pltpu.sync_copy(src_hbm.at[pl.ds(toff, CS)],
                                si0.at[pl.ds(0, CS)])
                pltpu.sync_copy(dst_hbm.at[pl.ds(toff, CS)],
                                di0.at[pl.ds(0, CS)])
                pltpu.async_copy(
                    x_hbm.at[si0.at[pl.ds(0, CS)]], rb0, sg0).wait()
                pltpu.sync_copy(rb0, acc_s.at[di0.at[pl.ds(0, CS)]],
                                add=True)

        plsc.subcore_barrier()
        obase = pl.multiple_of(cid * n_acc + base, 8)
        pltpu.sync_copy(acc_s.at[pl.ds(base, rows_pt)],
                        out_hbm.at[pl.ds(obase, rows_pt)])

    return prop


@functools.partial(
    pl.kernel,
    out_type=tuple(
        jax.ShapeDtypeStruct((NC * n_acc, 16), jnp.float32)
        for n_acc in (N_CNT_UI, N_CNT_UI, N_CNT_UU, N_CNT_UU)
    ),
    mesh=plsc.VectorSubcoreMesh(core_axis_name="c", subcore_axis_name="s"),
    compiler_params=pltpu.CompilerParams(use_tc_tiling_on_sc=False),
    scratch_types=[
        pltpu.VMEM_SHARED((N_CNT_UI, 16), jnp.float32),
        pltpu.VMEM_SHARED((N_CNT_UI, 16), jnp.float32),
        pltpu.VMEM_SHARED((N_CNT_UU, 16), jnp.float32),
        pltpu.VMEM_SHARED((N_CNT_UU, 16), jnp.float32),
        pltpu.VMEM((SUPD,), jnp.int32),
        pltpu.VMEM((SUPD,), jnp.int32),
        pltpu.VMEM((CD, 16), jnp.float32),
        pltpu.VMEM((160, 16), jnp.float32),
        pltpu.SemaphoreType.DMA,
    ],
)
def _sc_degrees(s_ui, d_ui, s_uu, d_uu, o0, o1, o2, o3,
                cnt0, cnt1, cnt2, cnt3, is_v, id_v, ones_v, zb, sem):
    sid = lax.axis_index("s")
    cid = lax.axis_index("c")
    wid = sid * NC + cid
    cnts = (cnt0, cnt1, cnt2, cnt3)
    outs = (o0, o1, o2, o3)
    sizes = (N_CNT_UI, N_CNT_UI, N_CNT_UU, N_CNT_UU)
    _fill(zb, 160, 16, 0.0)
    _fill(ones_v, CD, 16, 1.0)
    for cnt, n_acc in zip(cnts, sizes):
        rpt = n_acc // NS
        base = pl.multiple_of(sid * rpt, 8)
        for k in range(rpt // 160):
            pltpu.sync_copy(zb, cnt.at[pl.ds(base + k * 160, 160)])
    plsc.subcore_barrier()

    for src_hbm, dst_hbm, cnt_s, cnt_d, n_edges in (
        (s_ui, d_ui, cnt0, cnt1, E_UI_K),
        (s_uu, d_uu, cnt2, cnt3, E_UU_K),
    ):
        n_sup = n_edges // SUPD
        tail_off = n_sup * SUPD
        tail = n_edges - tail_off
        iters = -(-n_sup // NW)

        def body(it, _, src_hbm=src_hbm, dst_hbm=dst_hbm,
                 cnt_s=cnt_s, cnt_d=cnt_d, n_sup=n_sup):
            sup = wid + it * NW

            @pl.when(sup < n_sup)
            def _():
                off = pl.multiple_of(sup * SUPD, SUPD)
                pltpu.sync_copy(src_hbm.at[pl.ds(off, SUPD)], is_v)
                pltpu.sync_copy(dst_hbm.at[pl.ds(off, SUPD)], id_v)
                ds = []
                for k in range(SUPD // CD):
                    ds.append(pltpu.async_copy(
                        ones_v, cnt_s.at[is_v.at[pl.ds(k * CD, CD)]],
                        sem, add=True))
                    ds.append(pltpu.async_copy(
                        ones_v, cnt_d.at[id_v.at[pl.ds(k * CD, CD)]],
                        sem, add=True))
                for d in ds:
                    d.wait()
            return 0

        lax.fori_loop(0, iters, body, 0)

        @pl.when(wid == 0)
        def _(src_hbm=src_hbm, dst_hbm=dst_hbm, cnt_s=cnt_s, cnt_d=cnt_d,
              tail_off=tail_off, tail=tail):
            pltpu.sync_copy(src_hbm.at[pl.ds(tail_off, tail)],
                            is_v.at[pl.ds(0, tail)])
            pltpu.sync_copy(dst_hbm.at[pl.ds(tail_off, tail)],
                            id_v.at[pl.ds(0, tail)])
            pltpu.sync_copy(ones_v.at[pl.ds(0, tail)],
                            cnt_s.at[is_v.at[pl.ds(0, tail)]], add=True)
            pltpu.sync_copy(ones_v.at[pl.ds(0, tail)],
                            cnt_d.at[id_v.at[pl.ds(0, tail)]], add=True)

    plsc.subcore_barrier()
    for cnt, out, n_acc in zip(cnts, outs, sizes):
        rpt = n_acc // NS
        base = pl.multiple_of(sid * rpt, 8)
        obase = pl.multiple_of(cid * n_acc + base, 8)
        pltpu.sync_copy(cnt.at[pl.ds(base, rpt)],
                        out.at[pl.ds(obase, rpt)])


def _tc_prep(n_nodes, n_cnt, r=1000):
    def body(x_ref, cs_ref, o_ref):
        cs = cs_ref[...]
        c = cs[0, :, 0:1] + cs[1, :, 0:1]
        o_ref[...] = x_ref[...] * lax.rsqrt(jnp.maximum(c, 1.0))

    return pl.pallas_call(
        body,
        grid=(n_nodes // r,),
        in_specs=[pl.BlockSpec((r, D), lambda i: (i, 0)),
                  pl.BlockSpec((2, r, 16), lambda i: (0, i, 0))],
        out_specs=pl.BlockSpec((r, D), lambda i: (i, 0)),
        out_shape=jax.ShapeDtypeStruct((n_nodes, D), jnp.float32),
    )


def _tc_layer(n_nodes, n_acc, n_cnt, r=1000):
    def body(p_ref, cs_ref, cd_ref, a_ref, ao_ref, xs_ref):
        p = p_ref[...]
        y = p[0] + p[1]
        cd = cd_ref[...]
        c_in = cd[0, :, 0:1] + cd[1, :, 0:1]
        y = y * lax.rsqrt(jnp.maximum(c_in, 1.0))
        y = jnp.where(y >= 0, y, 0.5 * y)
        ss = jnp.sum(y * y, axis=1, keepdims=True)
        ao_ref[...] = a_ref[...] + y / jnp.maximum(jnp.sqrt(ss), 1e-12)
        cs = cs_ref[...]
        c_out = cs[0, :, 0:1] + cs[1, :, 0:1]
        xs_ref[...] = y * lax.rsqrt(jnp.maximum(c_out, 1.0))

    return pl.pallas_call(
        body,
        grid=(n_nodes // r,),
        in_specs=[pl.BlockSpec((2, r, D), lambda i: (0, i, 0)),
                  pl.BlockSpec((2, r, 16), lambda i: (0, i, 0)),
                  pl.BlockSpec((2, r, 16), lambda i: (0, i, 0)),
                  pl.BlockSpec((r, D), lambda i: (i, 0))],
        out_specs=[pl.BlockSpec((r, D), lambda i: (i, 0))] * 2,
        out_shape=[jax.ShapeDtypeStruct((n_nodes, D), jnp.float32)] * 2,
    )


_prop_ui = _make_prop(N_ACC_UI, E_UI_K, CS_UI)
_prop_uu = _make_prop(N_ACC_UU, E_UU_K, CS_UU)
_prep_ui = _tc_prep(N_TOTAL_K, N_CNT_UI)
_prep_uu = _tc_prep(N_USER_K, N_CNT_UU)
_layer_ui = _tc_layer(N_TOTAL_K, N_ACC_UI, N_CNT_UI)
_layer_uu = _tc_layer(N_USER_K, N_ACC_UU, N_CNT_UU)


def kernel(user_emb, item_emb, edge_index_ui, edge_index_uu):
    su, du = edge_index_ui[0], edge_index_ui[1]
    sv, dv = edge_index_uu[0], edge_index_uu[1]

    c_su, c_du, c_sv, c_dv = _sc_degrees(su, du, sv, dv)
    c_su = c_su.reshape(NC, N_CNT_UI, 16)
    c_du = c_du.reshape(NC, N_CNT_UI, 16)
    c_sv = c_sv.reshape(NC, N_CNT_UU, 16)
    c_dv = c_dv.reshape(NC, N_CNT_UU, 16)

    x0 = jnp.concatenate([user_emb, item_emb], axis=0)
    xs = _prep_ui(x0, c_su)
    p = _prop_ui(xs, su, du).reshape(NC, N_ACC_UI, D)
    acc, xs = _layer_ui(p, c_su, c_du, x0)
    p = _prop_ui(xs, su, du).reshape(NC, N_ACC_UI, D)
    ui_out, _ = _layer_ui(p, c_su, c_du, acc)

    xs = _prep_uu(user_emb, c_sv)
    p = _prop_uu(xs, sv, dv).reshape(NC, N_ACC_UU, D)
    acc, xs = _layer_uu(p, c_sv, c_dv, user_emb)
    p = _prop_uu(xs, sv, dv).reshape(NC, N_ACC_UU, D)
    uu_out, _ = _layer_uu(p, c_sv, c_dv, acc)

    return (ui_out, uu_out)
